# baseline (speedup 1.0000x reference)
import jax
import jax.numpy as jnp
from jax import lax
from jax.experimental import pallas as pl
from jax.experimental.pallas import tpu as pltpu
from jax.experimental.pallas import tpu_sc as plsc

N = 10000
E = 320000
D = 128
HD = 64
B = 64
L = 3
STEPS = 3
EPS = 1e-7

NT = 16
EPT = E // NT
KC = 50
IB = 10
NB = 40
NZT = 10
NPZ = N // NZT
NBLK = 25
BR = N // NBLK


def _sc_agg_body(h2, sd5, t16, c128, zer, out,
                 sda, sdb, gA0, gA1, gA2, gA3, ub0, ub1, t_v, c_v, acc,
                 sem_i, sem_g, sem_s):
    c = lax.axis_index("c")
    s = lax.axis_index("s")
    hv = h2.at[c]
    gbs = [gA0, gA1, gA2, gA3]
    ubs = [ub0, ub1]

    pltpu.sync_copy(t16, t_v)
    pltpu.sync_copy(c128, c_v)
    t = t_v[...]
    cvs = [c_v[pl.ds(c * HD + f * 16, 16)] for f in range(4)]

    @pl.when(s < NZT)
    def _():
        pltpu.sync_copy(zer, acc.at[pl.ds(s * NPZ, NPZ)])

    pltpu.sync_copy(sd5.at[0, s, 0], sda.at[0])
    pltpu.sync_copy(sd5.at[1, s, 0], sda.at[1])
    pltpu.async_copy(sd5.at[0, s, 1], sdb.at[0], sem_i)
    pltpu.async_copy(sd5.at[1, s, 1], sdb.at[1], sem_i)

    plsc.subcore_barrier()

    def compute(gb, ub):
        @plsc.parallel_loop(0, KC, unroll=5)
        def _edge(k):
            for f in range(4):
                g = gb[k, pl.ds(f * 16, 16)]
                m = jnp.maximum(g, 0.0) + EPS
                a = jnp.exp(m * t - cvs[f])
                ub[k, pl.ds(f * 16, 16)] = a
                ub[k, pl.ds(HD + f * 16, 16)] = a * m

    def g_start(sd, jj, u):
        pltpu.async_copy(hv.at[sd.at[0, jj]], gbs[u], sem_g)

    def g_wait(sd, jj, u):
        pltpu.make_async_copy(hv.at[sd.at[0, jj]], gbs[u], sem_g).wait()

    def s_wait():
        pltpu.make_async_copy(ub0, acc.at[sda.at[1, 0]], sem_s).wait()

    def wait_i():
        pltpu.make_async_copy(sd5.at[0, s, 0], sdb.at[0], sem_i).wait()
        pltpu.make_async_copy(sd5.at[1, s, 0], sdb.at[1], sem_i).wait()

    def substep(sd, jj, u, wq, nxt, nxt_guard):
        g_wait(sd, jj, u)

        @pl.when(wq >= 2)
        def _():
            s_wait()

        compute(gbs[u], ubs[u % 2])
        pltpu.async_copy(ubs[u % 2], acc.at[sd.at[1, jj]], sem_s, add=True)
        if nxt is not None:
            nsd, njj = nxt
            if nxt_guard is None:
                g_start(nsd, njj, u)
            else:
                @pl.when(nxt_guard)
                def _():
                    g_start(nsd, njj, u)

    def run_block(sd, sdn, tail_guard):
        @pl.loop(0, NB - 4, step=4)
        def _inner(j):
            for u in range(4):
                substep(sd, j + u, u, j + u, (sd, j + u + 4), None)

        for u in range(4):
            substep(sd, NB - 4 + u, u, NB - 4 + u, (sdn, u), tail_guard)

        s_wait()
        s_wait()

    for u in range(4):
        g_start(sda, u, u)

    @pl.loop(0, IB, step=2)
    def _pair(bb):
        wait_i()
        run_block(sda, sdb, None)

        @pl.when(bb + 2 < IB)
        def _():
            pltpu.async_copy(sd5.at[0, s, bb + 2], sda.at[0], sem_i)
            pltpu.async_copy(sd5.at[1, s, bb + 2], sda.at[1], sem_i)
            wait_i()

        run_block(sdb, sda, bb + 2 < IB)

        @pl.when(bb + 3 < IB)
        def _():
            pltpu.async_copy(sd5.at[0, s, bb + 3], sdb.at[0], sem_i)
            pltpu.async_copy(sd5.at[1, s, bb + 3], sdb.at[1], sem_i)

    plsc.subcore_barrier()

    @pl.when(s < NZT)
    def _():
        pltpu.sync_copy(acc.at[pl.ds(s * NPZ, NPZ)],
                        out.at[c, pl.ds(s * NPZ, NPZ)])


def _sc_aggregate(h2, sd5, t16, c128, zer):
    mesh = plsc.VectorSubcoreMesh(core_axis_name="c", subcore_axis_name="s")
    return pl.kernel(
        _sc_agg_body,
        out_type=jax.ShapeDtypeStruct((2, N, D), jnp.float32),
        mesh=mesh,
        compiler_params=pltpu.CompilerParams(use_tc_tiling_on_sc=False),
        scratch_types=[
            pltpu.VMEM((2, NB, KC), jnp.int32),
            pltpu.VMEM((2, NB, KC), jnp.int32),
            pltpu.VMEM((KC, HD), jnp.float32),
            pltpu.VMEM((KC, HD), jnp.float32),
            pltpu.VMEM((KC, HD), jnp.float32),
            pltpu.VMEM((KC, HD), jnp.float32),
            pltpu.VMEM((KC, D), jnp.float32),
            pltpu.VMEM((KC, D), jnp.float32),
            pltpu.VMEM((16,), jnp.float32),
            pltpu.VMEM((D,), jnp.float32),
            pltpu.VMEM_SHARED((N, D), jnp.float32),
            pltpu.SemaphoreType.DMA,
            pltpu.SemaphoreType.DMA,
            pltpu.SemaphoreType.DMA,
        ],
    )(h2, sd5, t16, c128, zer)


def _tc_prologue_body(x_ref, h2_ref, cmax_ref, cmin_ref):
    i = pl.program_id(0)
    xb = x_ref[...]
    h2_ref[0] = xb[:, :HD]
    h2_ref[1] = xb[:, HD:]
    zb = jnp.maximum(xb, 0.0) + EPS
    bmax = jnp.max(zb, axis=0, keepdims=True)
    bmin = jnp.min(zb, axis=0, keepdims=True)

    @pl.when(i == 0)
    def _():
        cmax_ref[...] = bmax
        cmin_ref[...] = bmin

    @pl.when(i != 0)
    def _():
        cmax_ref[...] = jnp.maximum(cmax_ref[...], bmax)
        cmin_ref[...] = jnp.minimum(cmin_ref[...], bmin)


def _tc_prologue(x):
    return pl.pallas_call(
        _tc_prologue_body,
        grid=(NBLK,),
        in_specs=[pl.BlockSpec((BR, D), lambda i: (i, 0))],
        out_specs=[
            pl.BlockSpec((2, BR, HD), lambda i: (0, i, 0)),
            pl.BlockSpec((1, D), lambda i: (0, 0)),
            pl.BlockSpec((1, D), lambda i: (0, 0)),
        ],
        out_shape=[
            jax.ShapeDtypeStruct((2, N, HD), jnp.float32),
            jax.ShapeDtypeStruct((1, D), jnp.float32),
            jax.ShapeDtypeStruct((1, D), jnp.float32),
        ],
    )(x)


def _tc_layer_body(parts_ref, h2_ref, w1_ref, b1_ref,
                   g_ref, be_ref, w2_ref, b2_ref, s_ref,
                   ho2_ref, cmax_ref, cmin_ref):
    i = pl.program_id(0)
    den = jnp.concatenate([parts_ref[0, :, :HD], parts_ref[1, :, :HD]], axis=1)
    num = jnp.concatenate([parts_ref[0, :, HD:], parts_ref[1, :, HD:]], axis=1)
    agg = num / jnp.maximum(den, 1e-16)
    h = jnp.concatenate([h2_ref[0], h2_ref[1]], axis=1)
    anorm = jnp.sqrt(jnp.sum(agg * agg, axis=1, keepdims=True))
    agg_dir = agg / jnp.maximum(anorm, 1e-12)
    xnorm = jnp.sqrt(jnp.sum(h * h, axis=1, keepdims=True))
    outv = h + agg_dir * xnorm * s_ref[0, 0]
    y = jnp.dot(outv, w1_ref[...], preferred_element_type=jnp.float32) + b1_ref[...]
    mu = jnp.mean(y, axis=1, keepdims=True)
    var = jnp.mean((y - mu) * (y - mu), axis=1, keepdims=True)
    y = (y - mu) / jnp.sqrt(var + 1e-5) * g_ref[...] + be_ref[...]
    y = jnp.maximum(y, 0.0)
    hn = jnp.dot(y, w2_ref[...], preferred_element_type=jnp.float32) + b2_ref[...]
    ho2_ref[0] = hn[:, :HD]
    ho2_ref[1] = hn[:, HD:]
    zb = jnp.maximum(hn, 0.0) + EPS
    bmax = jnp.max(zb, axis=0, keepdims=True)
    bmin = jnp.min(zb, axis=0, keepdims=True)

    @pl.when(i == 0)
    def _():
        cmax_ref[...] = bmax
        cmin_ref[...] = bmin

    @pl.when(i != 0)
    def _():
        cmax_ref[...] = jnp.maximum(cmax_ref[...], bmax)
        cmin_ref[...] = jnp.minimum(cmin_ref[...], bmin)


def _tc_layer(parts, h2, W1i, b1i, gi, bei, W2i, b2i, si):
    return pl.pallas_call(
        _tc_layer_body,
        grid=(NBLK,),
        in_specs=[
            pl.BlockSpec((2, BR, D), lambda i: (0, i, 0)),
            pl.BlockSpec((2, BR, HD), lambda i: (0, i, 0)),
            pl.BlockSpec((D, 2 * D), lambda i: (0, 0)),
            pl.BlockSpec((1, 2 * D), lambda i: (0, 0)),
            pl.BlockSpec((1, 2 * D), lambda i: (0, 0)),
            pl.BlockSpec((1, 2 * D), lambda i: (0, 0)),
            pl.BlockSpec((2 * D, D), lambda i: (0, 0)),
            pl.BlockSpec((1, D), lambda i: (0, 0)),
            pl.BlockSpec((1, 1), lambda i: (0, 0)),
        ],
        out_specs=[
            pl.BlockSpec((2, BR, HD), lambda i: (0, i, 0)),
            pl.BlockSpec((1, D), lambda i: (0, 0)),
            pl.BlockSpec((1, D), lambda i: (0, 0)),
        ],
        out_shape=[
            jax.ShapeDtypeStruct((2, N, HD), jnp.float32),
            jax.ShapeDtypeStruct((1, D), jnp.float32),
            jax.ShapeDtypeStruct((1, D), jnp.float32),
        ],
    )(parts, h2, W1i, b1i.reshape(1, 2 * D), gi.reshape(1, 2 * D),
      bei.reshape(1, 2 * D), W2i, b2i.reshape(1, D), si.reshape(1, 1))


def _tc_s2s_body(ha_ref, hb_ref, hc_ref, bat_ref, wih_ref, whh_ref, lb_ref,
                 f1w_ref, f1b_ref, f2w_ref, f2b_ref, o_ref):
    bb = bat_ref[0, :]
    M = (bb[:, None] == lax.broadcasted_iota(jnp.int32, (N, B), 1)
         ).astype(jnp.float32)
    x0 = jnp.maximum(jnp.maximum(ha_ref[0], hb_ref[0]), hc_ref[0])
    x1 = jnp.maximum(jnp.maximum(ha_ref[1], hb_ref[1]), hc_ref[1])
    q = jnp.zeros((B, 2 * D), jnp.float32)
    hh = jnp.zeros((B, D), jnp.float32)
    cc = jnp.zeros((B, D), jnp.float32)
    for _ in range(STEPS):
        gates = (jnp.dot(q, wih_ref[...], preferred_element_type=jnp.float32)
                 + jnp.dot(hh, whh_ref[...], preferred_element_type=jnp.float32)
                 + lb_ref[...])
        gi = gates[:, :D]
        gf = gates[:, D:2 * D]
        gg = gates[:, 2 * D:3 * D]
        go = gates[:, 3 * D:]
        cc = jax.nn.sigmoid(gf) * cc + jax.nn.sigmoid(gi) * jnp.tanh(gg)
        hh = jax.nn.sigmoid(go) * jnp.tanh(cc)
        hhb0 = jnp.dot(M, hh[:, :HD], preferred_element_type=jnp.float32)
        hhb1 = jnp.dot(M, hh[:, HD:], preferred_element_type=jnp.float32)
        e = (jnp.sum(x0 * hhb0, axis=1, keepdims=True)
             + jnp.sum(x1 * hhb1, axis=1, keepdims=True))
        emax = jnp.max(jnp.where(M > 0.0, e, -jnp.inf), axis=0,
                       keepdims=True)
        rmax = jnp.max(jnp.where(M > 0.0, emax, -jnp.inf), axis=1,
                       keepdims=True)
        ae = jnp.exp(e - rmax)
        den = lax.dot_general(M, ae, (((0,), (0,)), ((), ())),
                              preferred_element_type=jnp.float32)
        num0 = lax.dot_general(M, ae * x0, (((0,), (0,)), ((), ())),
                               preferred_element_type=jnp.float32)
        num1 = lax.dot_general(M, ae * x1, (((0,), (0,)), ((), ())),
                               preferred_element_type=jnp.float32)
        r = jnp.concatenate([num0, num1], axis=1) / jnp.maximum(den, 1e-16)
        q = jnp.concatenate([hh, r], axis=1)
    o = jnp.maximum(
        jnp.dot(q, f1w_ref[...], preferred_element_type=jnp.float32)
        + f1b_ref[...], 0.0)
    o = jnp.maximum(
        jnp.dot(o, f2w_ref[...], preferred_element_type=jnp.float32)
        + f2b_ref[...], 0.0)
    o_ref[...] = o


def _tc_set2set(ha, hb, hc, batch1, lstm_Wih, lstm_Whh, lstm_b, fc1_W, fc1_b,
                fc2_W, fc2_b):
    return pl.pallas_call(
        _tc_s2s_body,
        out_shape=jax.ShapeDtypeStruct((B, D), jnp.float32),
    )(ha, hb, hc, batch1, lstm_Wih, lstm_Whh, lstm_b.reshape(1, 4 * D), fc1_W,
      fc1_b.reshape(1, D), fc2_W, fc2_b.reshape(1, D))


def kernel(x, edge_index, batch, conv_t, conv_s, W1, b1, ln_g, ln_b, W2, b2,
           lstm_Wih, lstm_Whh, lstm_b, fc1_W, fc1_b, fc2_W, fc2_b):
    sd5 = edge_index.reshape(2, NT, IB, NB, KC)
    batch1 = batch.reshape(1, N)
    zer = jnp.zeros((NPZ, D), jnp.float32)

    h2, cmax, cmin = _tc_prologue(x)
    hs = []
    for i in range(L):
        ti = conv_t[i]
        c128 = jnp.where(ti >= 0, ti * cmax, ti * cmin).reshape(D)
        t16 = jnp.full((16,), ti, jnp.float32)
        parts = _sc_aggregate(h2, sd5, t16, c128, zer)
        h2, cmax, cmin = _tc_layer(
            parts, h2, W1[i], b1[i], ln_g[i], ln_b[i], W2[i], b2[i],
            conv_s[i])
        hs.append(h2)
    return _tc_set2set(hs[0], hs[1], hs[2], batch1, lstm_Wih, lstm_Whh,
                       lstm_b, fc1_W, fc1_b, fc2_W, fc2_b)


# speedup vs baseline: 1.0132x; 1.0132x over previous
import jax
import jax.numpy as jnp
from jax import lax
from jax.experimental import pallas as pl
from jax.experimental.pallas import tpu as pltpu
from jax.experimental.pallas import tpu_sc as plsc

N = 10000
E = 320000
D = 128
HD = 64
B = 64
L = 3
STEPS = 3
EPS = 1e-7

NT = 16
EPT = E // NT
KC = 50
IB = 10
NB = 40
NZT = 10
NPZ = N // NZT
NBLK = 25
BR = N // NBLK


def _sc_agg_body(h2, sd5, t16, c128, zer, out,
                 sda, sdb, gA0, gA1, gA2, gA3, ub0, ub1, t_v, c_v, acc,
                 sem_i, sem_g, sem_s):
    c = lax.axis_index("c")
    s = lax.axis_index("s")
    hv = h2.at[c]
    gbs = [gA0, gA1, gA2, gA3]
    ubs = [ub0, ub1]

    pltpu.sync_copy(t16, t_v)
    pltpu.sync_copy(c128, c_v)
    t = t_v[...]
    cvs = [c_v[pl.ds(c * HD + f * 16, 16)] for f in range(4)]

    @pl.when(s < NZT)
    def _():
        pltpu.sync_copy(zer, acc.at[pl.ds(s * NPZ, NPZ)])

    pltpu.sync_copy(sd5.at[0, s, 0], sda.at[0])
    pltpu.sync_copy(sd5.at[1, s, 0], sda.at[1])
    pltpu.async_copy(sd5.at[0, s, 1], sdb.at[0], sem_i)
    pltpu.async_copy(sd5.at[1, s, 1], sdb.at[1], sem_i)

    plsc.subcore_barrier()

    def compute(gb, ub):
        @plsc.parallel_loop(0, KC, unroll=5)
        def _edge(k):
            for f in range(4):
                g = gb[k, pl.ds(f * 16, 16)]
                m = jnp.maximum(g, 0.0) + EPS
                a = jnp.exp(m * t - cvs[f])
                ub[k, pl.ds(f * 16, 16)] = a
                ub[k, pl.ds(HD + f * 16, 16)] = a * m

    def g_start(sd, jj, u):
        pltpu.async_copy(hv.at[sd.at[0, jj]], gbs[u], sem_g)

    def g_wait(sd, jj, u):
        pltpu.make_async_copy(hv.at[sd.at[0, jj]], gbs[u], sem_g).wait()

    def s_wait():
        pltpu.make_async_copy(ub0, acc.at[sda.at[1, 0]], sem_s).wait()

    def wait_i():
        pltpu.make_async_copy(sd5.at[0, s, 0], sdb.at[0], sem_i).wait()
        pltpu.make_async_copy(sd5.at[1, s, 0], sdb.at[1], sem_i).wait()

    def substep(sd, jj, u, wq, nxt, nxt_guard):
        g_wait(sd, jj, u)

        @pl.when(wq >= 2)
        def _():
            s_wait()

        compute(gbs[u], ubs[u % 2])
        pltpu.async_copy(ubs[u % 2], acc.at[sd.at[1, jj]], sem_s, add=True)
        if nxt is not None:
            nsd, njj = nxt
            if nxt_guard is None:
                g_start(nsd, njj, u)
            else:
                @pl.when(nxt_guard)
                def _():
                    g_start(nsd, njj, u)

    def run_block(sd, sdn, tail_guard):
        @pl.loop(0, NB - 4, step=4)
        def _inner(j):
            for u in range(4):
                substep(sd, j + u, u, j + u, (sd, j + u + 4), None)

        for u in range(4):
            substep(sd, NB - 4 + u, u, NB - 4 + u, (sdn, u), tail_guard)

        s_wait()
        s_wait()

    for u in range(4):
        g_start(sda, u, u)

    @pl.loop(0, IB, step=2)
    def _pair(bb):
        wait_i()
        run_block(sda, sdb, None)

        @pl.when(bb + 2 < IB)
        def _():
            pltpu.async_copy(sd5.at[0, s, bb + 2], sda.at[0], sem_i)
            pltpu.async_copy(sd5.at[1, s, bb + 2], sda.at[1], sem_i)
            wait_i()

        run_block(sdb, sda, bb + 2 < IB)

        @pl.when(bb + 3 < IB)
        def _():
            pltpu.async_copy(sd5.at[0, s, bb + 3], sdb.at[0], sem_i)
            pltpu.async_copy(sd5.at[1, s, bb + 3], sdb.at[1], sem_i)

    plsc.subcore_barrier()

    @pl.when(s < NZT)
    def _():
        pltpu.sync_copy(acc.at[pl.ds(s * NPZ, NPZ)],
                        out.at[c, pl.ds(s * NPZ, NPZ)])


def _sc_aggregate(h2, sd5, t16, c128, zer):
    mesh = plsc.VectorSubcoreMesh(core_axis_name="c", subcore_axis_name="s")
    return pl.kernel(
        _sc_agg_body,
        out_type=jax.ShapeDtypeStruct((2, N, D), jnp.float32),
        mesh=mesh,
        compiler_params=pltpu.CompilerParams(use_tc_tiling_on_sc=False),
        scratch_types=[
            pltpu.VMEM((2, NB, KC), jnp.int32),
            pltpu.VMEM((2, NB, KC), jnp.int32),
            pltpu.VMEM((KC, HD), jnp.float32),
            pltpu.VMEM((KC, HD), jnp.float32),
            pltpu.VMEM((KC, HD), jnp.float32),
            pltpu.VMEM((KC, HD), jnp.float32),
            pltpu.VMEM((KC, D), jnp.float32),
            pltpu.VMEM((KC, D), jnp.float32),
            pltpu.VMEM((16,), jnp.float32),
            pltpu.VMEM((D,), jnp.float32),
            pltpu.VMEM_SHARED((N, D), jnp.float32),
            pltpu.SemaphoreType.DMA,
            pltpu.SemaphoreType.DMA,
            pltpu.SemaphoreType.DMA,
        ],
    )(h2, sd5, t16, c128, zer)


def _tc_prologue_body(x_ref, h2_ref, cmax_ref, cmin_ref):
    i = pl.program_id(0)
    xb = x_ref[...]
    h2_ref[0] = xb[:, :HD]
    h2_ref[1] = xb[:, HD:]
    zb = jnp.maximum(xb, 0.0) + EPS
    bmax = jnp.max(zb, axis=0, keepdims=True)
    bmin = jnp.min(zb, axis=0, keepdims=True)

    @pl.when(i == 0)
    def _():
        cmax_ref[...] = bmax
        cmin_ref[...] = bmin

    @pl.when(i != 0)
    def _():
        cmax_ref[...] = jnp.maximum(cmax_ref[...], bmax)
        cmin_ref[...] = jnp.minimum(cmin_ref[...], bmin)


def _tc_prologue(x):
    return pl.pallas_call(
        _tc_prologue_body,
        grid=(NBLK,),
        in_specs=[pl.BlockSpec((BR, D), lambda i: (i, 0))],
        out_specs=[
            pl.BlockSpec((2, BR, HD), lambda i: (0, i, 0)),
            pl.BlockSpec((1, D), lambda i: (0, 0)),
            pl.BlockSpec((1, D), lambda i: (0, 0)),
        ],
        out_shape=[
            jax.ShapeDtypeStruct((2, N, HD), jnp.float32),
            jax.ShapeDtypeStruct((1, D), jnp.float32),
            jax.ShapeDtypeStruct((1, D), jnp.float32),
        ],
    )(x)


def _tc_layer_body(parts_ref, h2_ref, jk_ref, w1_ref, b1_ref,
                   g_ref, be_ref, w2_ref, b2_ref, s_ref,
                   ho2_ref, jko_ref, cmax_ref, cmin_ref):
    i = pl.program_id(0)
    den = jnp.concatenate([parts_ref[0, :, :HD], parts_ref[1, :, :HD]], axis=1)
    num = jnp.concatenate([parts_ref[0, :, HD:], parts_ref[1, :, HD:]], axis=1)
    agg = num / jnp.maximum(den, 1e-16)
    h = jnp.concatenate([h2_ref[0], h2_ref[1]], axis=1)
    anorm = jnp.sqrt(jnp.sum(agg * agg, axis=1, keepdims=True))
    agg_dir = agg / jnp.maximum(anorm, 1e-12)
    xnorm = jnp.sqrt(jnp.sum(h * h, axis=1, keepdims=True))
    outv = h + agg_dir * xnorm * s_ref[0, 0]
    y = jnp.dot(outv, w1_ref[...], preferred_element_type=jnp.float32) + b1_ref[...]
    mu = jnp.mean(y, axis=1, keepdims=True)
    var = jnp.mean((y - mu) * (y - mu), axis=1, keepdims=True)
    y = (y - mu) / jnp.sqrt(var + 1e-5) * g_ref[...] + be_ref[...]
    y = jnp.maximum(y, 0.0)
    hn = jnp.dot(y, w2_ref[...], preferred_element_type=jnp.float32) + b2_ref[...]
    ho2_ref[0] = hn[:, :HD]
    ho2_ref[1] = hn[:, HD:]
    jko_ref[...] = jnp.maximum(jk_ref[...], hn)
    zb = jnp.maximum(hn, 0.0) + EPS
    bmax = jnp.max(zb, axis=0, keepdims=True)
    bmin = jnp.min(zb, axis=0, keepdims=True)

    @pl.when(i == 0)
    def _():
        cmax_ref[...] = bmax
        cmin_ref[...] = bmin

    @pl.when(i != 0)
    def _():
        cmax_ref[...] = jnp.maximum(cmax_ref[...], bmax)
        cmin_ref[...] = jnp.minimum(cmin_ref[...], bmin)


def _tc_layer(parts, h2, jk, W1i, b1i, gi, bei, W2i, b2i, si):
    return pl.pallas_call(
        _tc_layer_body,
        grid=(NBLK,),
        in_specs=[
            pl.BlockSpec((2, BR, D), lambda i: (0, i, 0)),
            pl.BlockSpec((2, BR, HD), lambda i: (0, i, 0)),
            pl.BlockSpec((BR, D), lambda i: (i, 0)),
            pl.BlockSpec((D, 2 * D), lambda i: (0, 0)),
            pl.BlockSpec((1, 2 * D), lambda i: (0, 0)),
            pl.BlockSpec((1, 2 * D), lambda i: (0, 0)),
            pl.BlockSpec((1, 2 * D), lambda i: (0, 0)),
            pl.BlockSpec((2 * D, D), lambda i: (0, 0)),
            pl.BlockSpec((1, D), lambda i: (0, 0)),
            pl.BlockSpec((1, 1), lambda i: (0, 0)),
        ],
        out_specs=[
            pl.BlockSpec((2, BR, HD), lambda i: (0, i, 0)),
            pl.BlockSpec((BR, D), lambda i: (i, 0)),
            pl.BlockSpec((1, D), lambda i: (0, 0)),
            pl.BlockSpec((1, D), lambda i: (0, 0)),
        ],
        out_shape=[
            jax.ShapeDtypeStruct((2, N, HD), jnp.float32),
            jax.ShapeDtypeStruct((N, D), jnp.float32),
            jax.ShapeDtypeStruct((1, D), jnp.float32),
            jax.ShapeDtypeStruct((1, D), jnp.float32),
        ],
    )(parts, h2, jk, W1i, b1i.reshape(1, 2 * D), gi.reshape(1, 2 * D),
      bei.reshape(1, 2 * D), W2i, b2i.reshape(1, D), si.reshape(1, 1))


def _tc_s2s_body(x1_ref, bat_ref, wih_ref, whh_ref, lb_ref,
                 f1w_ref, f1b_ref, f2w_ref, f2b_ref, o_ref):
    bb = bat_ref[0, :]
    M = (bb[:, None] == lax.broadcasted_iota(jnp.int32, (N, B), 1)
         ).astype(jnp.float32)
    x1 = x1_ref[...]
    q = jnp.zeros((B, 2 * D), jnp.float32)
    hh = jnp.zeros((B, D), jnp.float32)
    cc = jnp.zeros((B, D), jnp.float32)
    for _ in range(STEPS):
        gates = (jnp.dot(q, wih_ref[...], preferred_element_type=jnp.float32)
                 + jnp.dot(hh, whh_ref[...], preferred_element_type=jnp.float32)
                 + lb_ref[...])
        gi = gates[:, :D]
        gf = gates[:, D:2 * D]
        gg = gates[:, 2 * D:3 * D]
        go = gates[:, 3 * D:]
        cc = jax.nn.sigmoid(gf) * cc + jax.nn.sigmoid(gi) * jnp.tanh(gg)
        hh = jax.nn.sigmoid(go) * jnp.tanh(cc)
        hhb = jnp.dot(M, hh, preferred_element_type=jnp.float32)
        e = jnp.sum(x1 * hhb, axis=1, keepdims=True)
        emax = jnp.max(jnp.where(M > 0.0, e, -jnp.inf), axis=0,
                       keepdims=True)
        rmax = jnp.max(jnp.where(M > 0.0, emax, -jnp.inf), axis=1,
                       keepdims=True)
        ae = jnp.exp(e - rmax)
        den = lax.dot_general(M, ae, (((0,), (0,)), ((), ())),
                              preferred_element_type=jnp.float32)
        num = lax.dot_general(M, ae * x1, (((0,), (0,)), ((), ())),
                              preferred_element_type=jnp.float32)
        r = num / jnp.maximum(den, 1e-16)
        q = jnp.concatenate([hh, r], axis=1)
    o = jnp.maximum(
        jnp.dot(q, f1w_ref[...], preferred_element_type=jnp.float32)
        + f1b_ref[...], 0.0)
    o = jnp.maximum(
        jnp.dot(o, f2w_ref[...], preferred_element_type=jnp.float32)
        + f2b_ref[...], 0.0)
    o_ref[...] = o


def _tc_set2set(jk, batch1, lstm_Wih, lstm_Whh, lstm_b, fc1_W, fc1_b,
                fc2_W, fc2_b):
    return pl.pallas_call(
        _tc_s2s_body,
        out_shape=jax.ShapeDtypeStruct((B, D), jnp.float32),
    )(jk, batch1, lstm_Wih, lstm_Whh, lstm_b.reshape(1, 4 * D), fc1_W,
      fc1_b.reshape(1, D), fc2_W, fc2_b.reshape(1, D))


def kernel(x, edge_index, batch, conv_t, conv_s, W1, b1, ln_g, ln_b, W2, b2,
           lstm_Wih, lstm_Whh, lstm_b, fc1_W, fc1_b, fc2_W, fc2_b):
    sd5 = edge_index.reshape(2, NT, IB, NB, KC)
    batch1 = batch.reshape(1, N)
    zer = jnp.zeros((NPZ, D), jnp.float32)

    h2, cmax, cmin = _tc_prologue(x)
    jk = jnp.full((N, D), -jnp.inf, jnp.float32)
    for i in range(L):
        ti = conv_t[i]
        c128 = jnp.where(ti >= 0, ti * cmax, ti * cmin).reshape(D)
        t16 = jnp.full((16,), ti, jnp.float32)
        parts = _sc_aggregate(h2, sd5, t16, c128, zer)
        h2, jk, cmax, cmin = _tc_layer(
            parts, h2, jk, W1[i], b1[i], ln_g[i], ln_b[i], W2[i], b2[i],
            conv_s[i])
    return _tc_set2set(jk, batch1, lstm_Wih, lstm_Whh, lstm_b, fc1_W, fc1_b,
                       fc2_W, fc2_b)
